# Initial kernel scaffold; baseline (speedup 1.0000x reference)
import functools

import jax
import jax.numpy as jnp
from jax import lax
from jax.experimental import pallas as pl
from jax.experimental.pallas import tpu as pltpu
from jax.experimental.pallas import tpu_sc as plsc

N = 50000
D = 128
E = 250000

NC = 2
NS = 16
L = 16

NPASS = 4
UNITS = NC * NPASS
RNG = 6272
NPAD = UNITS * RNG
ACC_ROWS = RNG + 8

EP = 250112
EPT = EP // NS
VREGS = EPT // L

HALF = 25000
DEG_P = 25088
DEG_T = DEG_P // NS

MB = 98
RB = 512


def _vmesh():
    return plsc.VectorSubcoreMesh(
        core_axis_name="c", subcore_axis_name="s", num_cores=NC, num_subcores=NS
    )


def _deg_body(s0, d0, s1, d1, deg_out, idx_v, stage_v, acc0, acc1, acc2, acc3,
              ones_v):
    c = lax.axis_index("c")
    w = lax.axis_index("s")
    zeros = jnp.zeros((L,), jnp.float32)

    def zero_body(i, _):
        stage_v[pl.ds(i * L, L)] = zeros
        return 0
    lax.fori_loop(0, DEG_T // L, zero_body, 0)
    for acc in (acc0, acc1, acc2, acc3):
        pltpu.sync_copy(stage_v, acc.at[pl.ds(w * DEG_T, DEG_T)])
    ones_v[...] = jnp.ones((L,), jnp.float32)
    plsc.subcore_barrier()

    lo = c * HALF
    lane = lax.iota(jnp.int32, (L,))
    dump = HALF + (lane & 7)
    for arr, acc in ((s0, acc0), (d0, acc1), (s1, acc2), (d1, acc3)):
        pltpu.sync_copy(arr.at[pl.ds(w * EPT, EPT)], idx_v)

        def cnt_body(i, _, acc=acc):
            v = idx_v[pl.ds(i * L, L)] - lo
            ok = (v >= 0) & (v < HALF)
            iv = jnp.where(ok, v, dump)
            pltpu.sync_copy(ones_v, acc.at[iv], add=True)
            return 0
        lax.fori_loop(0, VREGS, cnt_body, 0)
    plsc.subcore_barrier()

    for a, acc in enumerate((acc0, acc1, acc2, acc3)):
        pltpu.sync_copy(acc.at[pl.ds(w * DEG_T, DEG_T)], stage_v)
        pltpu.sync_copy(stage_v, deg_out.at[a, c, pl.ds(w * DEG_T, DEG_T)])


def _sc_degrees(s0, d0, s1, d1):
    f = pl.kernel(
        _deg_body,
        out_type=jax.ShapeDtypeStruct((4, NC, DEG_P), jnp.float32),
        mesh=_vmesh(),
        scratch_types=[
            pltpu.VMEM((EPT,), jnp.int32),
            pltpu.VMEM((DEG_T,), jnp.float32),
            pltpu.VMEM_SHARED((DEG_P,), jnp.float32),
            pltpu.VMEM_SHARED((DEG_P,), jnp.float32),
            pltpu.VMEM_SHARED((DEG_P,), jnp.float32),
            pltpu.VMEM_SHARED((DEG_P,), jnp.float32),
            pltpu.VMEM((L,), jnp.float32),
        ],
    )
    return f(s0, d0, s1, d1)


def _agg_body(h0, h1, s0, d0, s1, d1, agg0, agg1,
              sidx, didx, csrc, cdst, rows, zbuf, dbuf, acc0, acc1):
    c = lax.axis_index("c")
    w = lax.axis_index("s")
    zeros = jnp.zeros((L,), jnp.float32)
    lane = lax.iota(jnp.int32, (L,))

    def zero_body(i, _):
        for jc in range(8):
            zbuf[i, pl.ds(jc * L, L)] = zeros
        return 0
    lax.fori_loop(0, 98, zero_body, 0)

    def one_pass(p, _):
        q = c * NPASS + p
        base = q * RNG

        for acc in (acc0, acc1):
            for k in range(4):
                pltpu.sync_copy(zbuf, acc.at[pl.ds(w * 392 + k * 98, 98)])
        plsc.subcore_barrier()

        for sarr, darr, acc, h in ((s0, d0, acc0, h0), (s1, d1, acc1, h1)):
            pltpu.sync_copy(sarr.at[pl.ds(w * EPT, EPT)], sidx)
            pltpu.sync_copy(darr.at[pl.ds(w * EPT, EPT)], didx)

            def compact_body(i, cur, acc=acc):
                s = sidx[pl.ds(i * L, L)]
                dloc = didx[pl.ds(i * L, L)] - base
                ok = (dloc >= 0) & (dloc < RNG)
                plsc.store_compressed(csrc.at[pl.ds(cur, L)], s, mask=ok)
                plsc.store_compressed(
                    cdst.at[pl.ds(cur, L)], jnp.where(ok, dloc, 0), mask=ok)
                return cur + jnp.sum(ok.astype(jnp.int32))
            cur = lax.fori_loop(0, VREGS, compact_body, jnp.int32(0))

            csrc[pl.ds(cur, L)] = jnp.full((L,), N, jnp.int32)
            cdst[pl.ds(cur, L)] = RNG + (lane & 7)
            nchunks = (cur + L - 1) // L

            def gs_body(j, _, acc=acc, h=h):
                iv = csrc[pl.ds(j * L, L)]
                pltpu.sync_copy(h.at[iv], rows)
                dv = cdst[pl.ds(j * L, L)]
                pltpu.sync_copy(rows, acc.at[dv], add=True)
                return 0
            lax.fori_loop(0, nchunks, gs_body, 0)
        plsc.subcore_barrier()

        for acc, agg in ((acc0, agg0), (acc1, agg1)):
            for k in range(4):
                pltpu.sync_copy(acc.at[pl.ds(w * 392 + k * 98, 98)], dbuf)
                pltpu.sync_copy(
                    dbuf, agg.at[pl.ds(base + w * 392 + k * 98, 98)])
        plsc.subcore_barrier()
        return 0

    lax.fori_loop(0, NPASS, one_pass, 0)


def _sc_aggregate(h0, h1, s0, d0, s1, d1):
    f = pl.kernel(
        _agg_body,
        out_type=(
            jax.ShapeDtypeStruct((NPAD, D), jnp.float32),
            jax.ShapeDtypeStruct((NPAD, D), jnp.float32),
        ),
        mesh=_vmesh(),
        scratch_types=[
            pltpu.VMEM((EPT,), jnp.int32),
            pltpu.VMEM((EPT,), jnp.int32),
            pltpu.VMEM((EPT + L,), jnp.int32),
            pltpu.VMEM((EPT + L,), jnp.int32),
            pltpu.VMEM((L, D), jnp.float32),
            pltpu.VMEM((98, D), jnp.float32),
            pltpu.VMEM((98, D), jnp.float32),
            pltpu.VMEM_SHARED((ACC_ROWS, D), jnp.float32),
            pltpu.VMEM_SHARED((ACC_ROWS, D), jnp.float32),
        ],
    )
    return f(h0, h1, s0, d0, s1, d1)


def _norms_body(deg_ref, out_ref):
    d = deg_ref[...]
    out_ref[...] = lax.rsqrt(jnp.where(d > 0.0, d, 1.0))


def _tc_norms(deg):
    return pl.pallas_call(
        _norms_body,
        out_shape=jax.ShapeDtypeStruct((4, NPAD), jnp.float32),
        grid=(8,),
        in_specs=[pl.BlockSpec((4, NPAD // 8), lambda i: (0, i))],
        out_specs=pl.BlockSpec((4, NPAD // 8), lambda i: (0, i)),
    )(deg)


def _mm_body(x_ref, ns_ref, w_ref, out_ref):
    scale = jnp.reshape(ns_ref[...], (RB, 1))
    xs = x_ref[...] * scale
    out_ref[...] = jnp.dot(xs, w_ref[...], preferred_element_type=jnp.float32)


def _tc_matmul(x, ns3, w):
    return pl.pallas_call(
        _mm_body,
        out_shape=jax.ShapeDtypeStruct((NPAD, D), jnp.float32),
        grid=(MB,),
        in_specs=[
            pl.BlockSpec((RB, D), lambda i: (i, 0)),
            pl.BlockSpec((1, RB, 1), lambda i: (i, 0, 0)),
            pl.BlockSpec((D, D), lambda i: (0, 0)),
        ],
        out_specs=pl.BlockSpec((RB, D), lambda i: (i, 0)),
    )(x, ns3, w)


def _comb_body(a0_ref, a1_ref, n0_ref, n1_ref, b0_ref, b1_ref, out_ref):
    n0 = jnp.reshape(n0_ref[...], (RB, 1))
    n1 = jnp.reshape(n1_ref[...], (RB, 1))
    b = b0_ref[...] + b1_ref[...]
    h = a0_ref[...] * n0 + a1_ref[...] * n1 + b
    out_ref[...] = jnp.maximum(h, 0.0)


def _tc_combine(a0, a1, n0, n1, b0, b1):
    return pl.pallas_call(
        _comb_body,
        out_shape=jax.ShapeDtypeStruct((N, D), jnp.float32),
        grid=(MB,),
        in_specs=[
            pl.BlockSpec((RB, D), lambda i: (i, 0)),
            pl.BlockSpec((RB, D), lambda i: (i, 0)),
            pl.BlockSpec((1, RB, 1), lambda i: (i, 0, 0)),
            pl.BlockSpec((1, RB, 1), lambda i: (i, 0, 0)),
            pl.BlockSpec((1, D), lambda i: (0, 0)),
            pl.BlockSpec((1, D), lambda i: (0, 0)),
        ],
        out_specs=pl.BlockSpec((RB, D), lambda i: (i, 0)),
    )(a0, a1, n0, n1, b0, b1)


def kernel(x, edge_index_r0, edge_index_r1, W1_0, b1_0, W1_1, b1_1,
           W2_0, b2_0, W2_1, b2_1):
    pad = jnp.full((EP - E,), N, jnp.int32)
    s0 = jnp.concatenate([edge_index_r0[0].astype(jnp.int32), pad])
    d0 = jnp.concatenate([edge_index_r0[1].astype(jnp.int32), pad])
    s1 = jnp.concatenate([edge_index_r1[0].astype(jnp.int32), pad])
    d1 = jnp.concatenate([edge_index_r1[1].astype(jnp.int32), pad])

    deg = _sc_degrees(s0, d0, s1, d1)
    deg_full = jnp.concatenate([deg[:, 0, :HALF], deg[:, 1, :HALF]], axis=1)
    deg_full = jnp.pad(deg_full, ((0, 0), (0, NPAD - N)))
    norms = _tc_norms(deg_full)
    ns0 = norms[0].reshape(MB, RB, 1)
    nd0 = norms[1].reshape(MB, RB, 1)
    ns1 = norms[2].reshape(MB, RB, 1)
    nd1 = norms[3].reshape(MB, RB, 1)
    b1_0r = b1_0.reshape(1, D)
    b1_1r = b1_1.reshape(1, D)
    b2_0r = b2_0.reshape(1, D)
    b2_1r = b2_1.reshape(1, D)

    h10 = _tc_matmul(x, ns0, W1_0)
    h11 = _tc_matmul(x, ns1, W1_1)
    a10, a11 = _sc_aggregate(h10, h11, s0, d0, s1, d1)
    x1 = _tc_combine(a10, a11, nd0, nd1, b1_0r, b1_1r)

    h20 = _tc_matmul(x1, ns0, W2_0)
    h21 = _tc_matmul(x1, ns1, W2_1)
    a20, a21 = _sc_aggregate(h20, h21, s0, d0, s1, d1)
    return _tc_combine(a20, a21, nd0, nd1, b2_0r, b2_1r)


# speedup vs baseline: 2.0072x; 2.0072x over previous
import jax
import jax.numpy as jnp
from jax import lax
from jax.experimental import pallas as pl
from jax.experimental.pallas import tpu as pltpu
from jax.experimental.pallas import tpu_sc as plsc

N = 50000
D = 128
E = 250000

NC = 2
NS = 16
L = 16

NPASS = 10
RNG = 5120
NPAD = NPASS * RNG
ACC_ROWS = RNG + 8
RPT = RNG // NS
DCH = 80

EP = 250112
EPT = EP // NS
VREGS = EPT // L
CAP = EPT + 2 * L

HALF = 25000
DEG_P = 25088
DEG_T = DEG_P // NS

MB = 100
RB = 512


def _vmesh():
    return plsc.VectorSubcoreMesh(
        core_axis_name="c", subcore_axis_name="s", num_cores=NC, num_subcores=NS
    )


def _deg_body(sd, deg_out, idx_v, stage_v, acc0, acc1, acc2, acc3, ones_v):
    c = lax.axis_index("c")
    w = lax.axis_index("s")
    zeros = jnp.zeros((L,), jnp.float32)

    def zero_body(i, _):
        stage_v[pl.ds(i * L, L)] = zeros
        return 0
    lax.fori_loop(0, DEG_T // L, zero_body, 0)
    for acc in (acc0, acc1, acc2, acc3):
        pltpu.sync_copy(stage_v, acc.at[pl.ds(w * DEG_T, DEG_T)])
    ones_v[...] = jnp.ones((L,), jnp.float32)
    plsc.subcore_barrier()

    lo = c * HALF
    lane = lax.iota(jnp.int32, L)
    dump = HALF + (lane & 7)
    for a, acc in enumerate((acc0, acc1, acc2, acc3)):
        pltpu.sync_copy(sd.at[pl.ds(a * EP + w * EPT, EPT)], idx_v)

        def cnt_body(i, _, acc=acc):
            v = idx_v[pl.ds(i * L, L)] - lo
            ok = (v >= 0) & (v < HALF)
            iv = jnp.where(ok, v, dump)
            pltpu.sync_copy(ones_v, acc.at[iv], add=True)
            return 0
        lax.fori_loop(0, VREGS, cnt_body, 0)
    plsc.subcore_barrier()

    for a, acc in enumerate((acc0, acc1, acc2, acc3)):
        pltpu.sync_copy(acc.at[pl.ds(w * DEG_T, DEG_T)], stage_v)
        pltpu.sync_copy(
            stage_v,
            deg_out.at[pl.ds((a * NC + c) * DEG_P + w * DEG_T, DEG_T)])


def _sc_degrees(sd):
    f = pl.kernel(
        _deg_body,
        out_type=jax.ShapeDtypeStruct((4 * NC * DEG_P,), jnp.float32),
        mesh=_vmesh(),
        scratch_types=[
            pltpu.VMEM((EPT,), jnp.int32),
            pltpu.VMEM((DEG_T,), jnp.float32),
            pltpu.VMEM_SHARED((DEG_P,), jnp.float32),
            pltpu.VMEM_SHARED((DEG_P,), jnp.float32),
            pltpu.VMEM_SHARED((DEG_P,), jnp.float32),
            pltpu.VMEM_SHARED((DEG_P,), jnp.float32),
            pltpu.VMEM((L,), jnp.float32),
        ],
    )
    return f(sd)


def _agg_body(hh, sd, agg, sidx, didx, csrc, cdst, rows, zbuf, dbuf, cntb, acc):
    c = lax.axis_index("c")
    w = lax.axis_index("s")
    zeros = jnp.zeros((L,), jnp.float32)
    lane = lax.iota(jnp.int32, L)
    hbase = c * NPAD
    sbase = 2 * c * EP + w * EPT

    def zbody(i, _):
        for jc in range(8):
            zbuf[i, pl.ds(jc * L, L)] = zeros
        return 0
    lax.fori_loop(0, DCH, zbody, 0)

    def one_pass(p, _):
        base = p * RNG

        for k in range(4):
            pltpu.sync_copy(zbuf, acc.at[pl.ds(w * RPT + k * DCH, DCH)])
        plsc.subcore_barrier()

        pltpu.sync_copy(sd.at[pl.ds(sbase, EPT)], sidx)
        pltpu.sync_copy(sd.at[pl.ds(sbase + EP, EPT)], didx)

        trash = EPT + L + lane

        def compact_body(i, cnt):
            s = sidx[pl.ds(i * L, L)]
            dloc = didx[pl.ds(i * L, L)] - base
            ok = (dloc >= 0) & (dloc < RNG)
            pos = jnp.where(ok, cnt * L + lane, trash)
            plsc.store_scatter(csrc, [pos], s)
            plsc.store_scatter(cdst, [pos], dloc)
            return cnt + jnp.where(ok, 1, 0)
        cnt = lax.fori_loop(0, VREGS, compact_body,
                            jnp.zeros((L,), jnp.int32))

        maxv = cnt
        for kk in (1, 2, 4, 8):
            maxv = jnp.maximum(
                maxv, maxv.at[lane ^ kk].get(mode='promise_in_bounds'))
        cntb[...] = maxv
        nchunks = cntb[pl.ds(0, L)][0]

        def gs_body(j, jv):
            m = jv < cnt
            iv = hbase + jnp.where(m, csrc[pl.ds(j * L, L)], N + (lane & 7))
            dv = jnp.where(m, cdst[pl.ds(j * L, L)], RNG + (lane & 7))
            pltpu.sync_copy(hh.at[iv], rows)
            pltpu.sync_copy(rows, acc.at[dv], add=True)
            return jv + 1
        lax.fori_loop(0, nchunks, gs_body, jnp.zeros((L,), jnp.int32))
        plsc.subcore_barrier()

        for k in range(4):
            pltpu.sync_copy(acc.at[pl.ds(w * RPT + k * DCH, DCH)], dbuf)
            pltpu.sync_copy(
                dbuf, agg.at[pl.ds(hbase + base + w * RPT + k * DCH, DCH)])
        plsc.subcore_barrier()
        return 0

    lax.fori_loop(0, NPASS, one_pass, 0)


def _sc_aggregate(hh, sd):
    f = pl.kernel(
        _agg_body,
        out_type=jax.ShapeDtypeStruct((2 * NPAD, D), jnp.float32),
        mesh=_vmesh(),
        compiler_params=pltpu.CompilerParams(needs_layout_passes=False),
        scratch_types=[
            pltpu.VMEM((EPT,), jnp.int32),
            pltpu.VMEM((EPT,), jnp.int32),
            pltpu.VMEM((CAP,), jnp.int32),
            pltpu.VMEM((CAP,), jnp.int32),
            pltpu.VMEM((L, D), jnp.float32),
            pltpu.VMEM((DCH, D), jnp.float32),
            pltpu.VMEM((DCH, D), jnp.float32),
            pltpu.VMEM((L,), jnp.int32),
            pltpu.VMEM_SHARED((ACC_ROWS, D), jnp.float32),
        ],
    )
    return f(hh, sd)


def _norms_body(deg_ref, out_ref):
    d = deg_ref[...]
    out_ref[...] = lax.rsqrt(jnp.where(d > 0.0, d, 1.0))


def _tc_norms(deg):
    return pl.pallas_call(
        _norms_body,
        out_shape=jax.ShapeDtypeStruct((4, NPAD), jnp.float32),
        grid=(8,),
        in_specs=[pl.BlockSpec((4, NPAD // 8), lambda i: (0, i))],
        out_specs=pl.BlockSpec((4, NPAD // 8), lambda i: (0, i)),
    )(deg)


def _mm_body(x_ref, ns_ref, w_ref, out_ref):
    scale = jnp.reshape(ns_ref[...], (RB, 1))
    xs = x_ref[...] * scale
    out_ref[...] = jnp.dot(
        xs, jnp.reshape(w_ref[...], (D, D)), preferred_element_type=jnp.float32)


def _tc_matmul2(x, ns_stack, w_stack):
    return pl.pallas_call(
        _mm_body,
        out_shape=jax.ShapeDtypeStruct((2 * NPAD, D), jnp.float32),
        grid=(2, 98),
        in_specs=[
            pl.BlockSpec((RB, D), lambda r, i: (i, 0)),
            pl.BlockSpec((1, 1, RB, 1), lambda r, i: (r, i, 0, 0)),
            pl.BlockSpec((1, D, D), lambda r, i: (r, 0, 0)),
        ],
        out_specs=pl.BlockSpec((RB, D), lambda r, i: (r * MB + i, 0)),
    )(x, ns_stack, w_stack)


def _comb_body(a0_ref, a1_ref, n0_ref, n1_ref, b_ref, out_ref):
    n0 = jnp.reshape(n0_ref[...], (RB, 1))
    n1 = jnp.reshape(n1_ref[...], (RB, 1))
    h = a0_ref[...] * n0 + a1_ref[...] * n1 + b_ref[...]
    out_ref[...] = jnp.maximum(h, 0.0)


def _tc_combine(agg, nd_stack, b, out_rows, nblocks):
    return pl.pallas_call(
        _comb_body,
        out_shape=jax.ShapeDtypeStruct((out_rows, D), jnp.float32),
        grid=(nblocks,),
        in_specs=[
            pl.BlockSpec((RB, D), lambda i: (i, 0)),
            pl.BlockSpec((RB, D), lambda i: (MB + i, 0)),
            pl.BlockSpec((1, 1, RB, 1), lambda i: (0, i, 0, 0)),
            pl.BlockSpec((1, 1, RB, 1), lambda i: (1, i, 0, 0)),
            pl.BlockSpec((1, D), lambda i: (0, 0)),
        ],
        out_specs=pl.BlockSpec((RB, D), lambda i: (i, 0)),
    )(agg, agg, nd_stack, nd_stack, b)


def kernel(x, edge_index_r0, edge_index_r1, W1_0, b1_0, W1_1, b1_1,
           W2_0, b2_0, W2_1, b2_1):
    pad = jnp.full((EP - E,), N, jnp.int32)
    sd = jnp.concatenate([
        edge_index_r0[0].astype(jnp.int32), pad,
        edge_index_r0[1].astype(jnp.int32), pad,
        edge_index_r1[0].astype(jnp.int32), pad,
        edge_index_r1[1].astype(jnp.int32), pad,
    ])

    deg = _sc_degrees(sd).reshape(4, NC, DEG_P)
    deg_full = jnp.concatenate([deg[:, 0, :HALF], deg[:, 1, :HALF]], axis=1)
    deg_full = jnp.pad(deg_full, ((0, 0), (0, NPAD - N)))
    norms = _tc_norms(deg_full)
    ns_stack = norms[0::2].reshape(2, MB, RB, 1)
    nd_stack = norms[1::2].reshape(2, MB, RB, 1)
    w1_stack = jnp.stack([W1_0, W1_1])
    w2_stack = jnp.stack([W2_0, W2_1])
    bias1 = (b1_0 + b1_1).reshape(1, D)
    bias2 = (b2_0 + b2_1).reshape(1, D)

    h1 = _tc_matmul2(x, ns_stack, w1_stack)
    a1 = _sc_aggregate(h1, sd)
    x1 = _tc_combine(a1, nd_stack, bias1, NPAD, MB)

    h2 = _tc_matmul2(x1, ns_stack, w2_stack)
    a2 = _sc_aggregate(h2, sd)
    return _tc_combine(a2, nd_stack, bias2, N, 98)
